# Initial kernel scaffold; baseline (speedup 1.0000x reference)
import jax
import jax.numpy as jnp
from jax.experimental import pallas as pl


def kernel(x, edge_index, W_src, b_src, W_dst, b_dst):
    raise NotImplementedError("write your pallas kernel here")


# speedup vs baseline: 1.8311x; 1.8311x over previous
import functools

import jax
import jax.numpy as jnp
from jax import lax
from jax.experimental import pallas as pl
from jax.experimental.pallas import tpu as pltpu
from jax.experimental.pallas import tpu_sc as plsc

N = 10000
NP = 10240
D = 128
E = 160000
EP = 163840
ALPHA1 = 0.5
ALPHA2 = 0.5
BETA1 = 0.7

C_F_SRC = BETA1 * ALPHA1
C_F_DST = BETA1 * (1.0 - ALPHA1)
C_S_SRC = (1.0 - BETA1) * ALPHA2
C_S_DST = (1.0 - BETA1) * (1.0 - ALPHA2)
C_BIAS = BETA1 * ALPHA1 + (1.0 - BETA1) * ALPHA2

NSUB = 16
STRIPE = NP * NP // NSUB
ZCH = 32768
NZ = STRIPE // ZCH
EPW = EP // NSUB
NB = EPW // 128

BI = 1280
KB = 1280
GI = NP // BI
GK = NP // KB
R2 = 320


def _sc_build_adjacency(rows, cols):
  mesh = plsc.VectorSubcoreMesh(
      core_axis_name="c", subcore_axis_name="s", num_cores=1)

  @functools.partial(
      pl.kernel,
      out_type=jax.ShapeDtypeStruct((NP * NP,), jnp.float32),
      mesh=mesh,
      scratch_types=[
          pltpu.VMEM((ZCH,), jnp.float32),
          pltpu.VMEM((128,), jnp.int32),
          pltpu.VMEM((128,), jnp.int32),
          pltpu.VMEM((128,), jnp.int32),
          pltpu.VMEM((128,), jnp.float32),
          pltpu.SemaphoreType.DMA,
      ],
  )
  def build(rows_hbm, cols_hbm, a_hbm, zbuf, rbuf, cbuf, ibuf, ones, sem):
    wid = lax.axis_index("s")

    def zfill(i, c):
      zbuf[pl.ds(i * 16, 16)] = jnp.zeros((16,), jnp.float32)
      return c
    lax.fori_loop(0, ZCH // 16, zfill, 0)

    def ofill(i, c):
      ones[pl.ds(i * 16, 16)] = jnp.ones((16,), jnp.float32)
      return c
    lax.fori_loop(0, 8, ofill, 0)

    base0 = wid * STRIPE

    def memset(j, c):
      pltpu.sync_copy(zbuf, a_hbm.at[pl.ds(base0 + j * ZCH, ZCH)])
      return c
    lax.fori_loop(0, NZ, memset, 0)

    plsc.subcore_barrier()

    ebase = wid * EPW

    def scat(b, c):
      s = ebase + b * 128
      pltpu.sync_copy(rows_hbm.at[pl.ds(s, 128)], rbuf)
      pltpu.sync_copy(cols_hbm.at[pl.ds(s, 128)], cbuf)
      for i in range(8):
        rv = rbuf[pl.ds(i * 16, 16)]
        cv = cbuf[pl.ds(i * 16, 16)]
        ibuf[pl.ds(i * 16, 16)] = rv * NP + cv
      pltpu.async_copy(ones, a_hbm.at[ibuf], sem).wait()
      return c
    lax.fori_loop(0, NB, scat, 0)

  return build(rows, cols)


def _k2_cast_degrees(a):
  def body(a_ref, abf_ref, od_ref, idr_ref):
    blk = a_ref[...]
    abf_ref[...] = blk.astype(jnp.bfloat16)
    od_ref[...] = jnp.sum(blk, axis=1, keepdims=True)
    part = jnp.sum(blk, axis=0, keepdims=True)

    @pl.when(pl.program_id(0) == 0)
    def _():
      idr_ref[...] = part

    @pl.when(pl.program_id(0) != 0)
    def _():
      idr_ref[...] += part

  return pl.pallas_call(
      body,
      grid=(NP // R2,),
      in_specs=[pl.BlockSpec((R2, NP), lambda i: (i, 0))],
      out_specs=[
          pl.BlockSpec((R2, NP), lambda i: (i, 0)),
          pl.BlockSpec((R2, 1), lambda i: (i, 0)),
          pl.BlockSpec((1, NP), lambda i: (0, 0)),
      ],
      out_shape=[
          jax.ShapeDtypeStruct((NP, NP), jnp.bfloat16),
          jax.ShapeDtypeStruct((NP, 1), jnp.float32),
          jax.ShapeDtypeStruct((1, NP), jnp.float32),
      ],
      compiler_params=pltpu.CompilerParams(
          dimension_semantics=("arbitrary",)),
  )(a)


def _k3_product(abf, transposed):
  def body(l_ref, r_ref, m_ref, q_ref, rs_ref, cs_ref, acc):
    i = pl.program_id(0)
    j = pl.program_id(1)
    k = pl.program_id(2)

    @pl.when(k == 0)
    def _():
      acc[...] = jnp.zeros_like(acc)

    if transposed:
      dn = (((0,), (0,)), ((), ()))
    else:
      dn = (((1,), (1,)), ((), ()))
    acc[...] += lax.dot_general(l_ref[...], r_ref[...], dn,
                                preferred_element_type=jnp.float32)

    @pl.when(k == GK - 1)
    def _():
      p = acc[...]
      amask = m_ref[...] > 0
      ii = lax.broadcasted_iota(jnp.int32, (BI, BI), 0)
      jj = lax.broadcasted_iota(jnp.int32, (BI, BI), 1)
      diag = (ii == jj) & (i == j)
      q = jnp.where(amask | diag, 0.0, p)
      q_ref[...] = q.astype(jnp.bfloat16)
      rpart = jnp.sum(q, axis=1, keepdims=True)

      @pl.when(j == 0)
      def _():
        rs_ref[...] = rpart

      @pl.when(j != 0)
      def _():
        rs_ref[...] += rpart

      cs_ref[...] = jnp.sum(q, axis=0, keepdims=True)[None]

  if transposed:
    lhs_spec = pl.BlockSpec((KB, BI), lambda i, j, k: (k, i))
    rhs_spec = pl.BlockSpec((KB, BI), lambda i, j, k: (k, j))
  else:
    lhs_spec = pl.BlockSpec((BI, KB), lambda i, j, k: (i, k))
    rhs_spec = pl.BlockSpec((BI, KB), lambda i, j, k: (j, k))

  return pl.pallas_call(
      body,
      grid=(GI, GI, GK),
      in_specs=[
          lhs_spec,
          rhs_spec,
          pl.BlockSpec((BI, BI), lambda i, j, k: (i, j)),
      ],
      out_specs=[
          pl.BlockSpec((BI, BI), lambda i, j, k: (i, j)),
          pl.BlockSpec((BI, 1), lambda i, j, k: (i, 0)),
          pl.BlockSpec((1, 1, BI), lambda i, j, k: (i, 0, j)),
      ],
      out_shape=[
          jax.ShapeDtypeStruct((NP, NP), jnp.bfloat16),
          jax.ShapeDtypeStruct((NP, 1), jnp.float32),
          jax.ShapeDtypeStruct((GI, 1, NP), jnp.float32),
      ],
      scratch_shapes=[pltpu.VMEM((BI, BI), jnp.float32)],
      compiler_params=pltpu.CompilerParams(
          dimension_semantics=("arbitrary", "arbitrary", "arbitrary")),
  )(abf, abf, abf)


def _rsqrt0(v):
  return jnp.where(v > 0, lax.rsqrt(v), 0.0)


def _k4_aggregate(abf, qo, qi, xp, odeg, idegc, rso, csop, rsi, csip,
                  w_src, w_dst, b_src, b_dst):
  def body(a_ik, a_ki, qo_ik, qi_ki, xk, idc_k, od_k, cso_k, csi_i,
           od_i, idc_i, rso_i, rsi_k, ws, wd, bs, bd,
           out_ref, acc1, acc2, acc3, acc4):
    k = pl.program_id(1)

    @pl.when(k == 0)
    def _():
      acc1[...] = jnp.zeros_like(acc1)
      acc2[...] = jnp.zeros_like(acc2)
      acc3[...] = jnp.zeros_like(acc3)
      acc4[...] = jnp.zeros_like(acc4)

    ones_g = jnp.ones((GI, 1), jnp.float32)
    dn_t = (((0,), (0,)), ((), ()))
    dn_s = (((1,), (0,)), ((), ()))

    xb = xk[...]
    xs1 = (_rsqrt0(idc_k[...]) * xb).astype(jnp.bfloat16)
    xs2 = (_rsqrt0(od_k[...]) * xb).astype(jnp.bfloat16)
    c3 = _rsqrt0(lax.dot_general(cso_k[...], ones_g, dn_t,
                                 preferred_element_type=jnp.float32))
    xs3 = (c3 * xb).astype(jnp.bfloat16)
    c4 = _rsqrt0(rsi_k[...])
    xs4 = (c4 * xb).astype(jnp.bfloat16)

    acc1[...] += lax.dot_general(a_ik[...], xs1, dn_s,
                                 preferred_element_type=jnp.float32)
    acc2[...] += lax.dot_general(a_ki[...], xs2, dn_t,
                                 preferred_element_type=jnp.float32)
    acc3[...] += lax.dot_general(qo_ik[...], xs3, dn_s,
                                 preferred_element_type=jnp.float32)
    acc4[...] += lax.dot_general(qi_ki[...], xs4, dn_t,
                                 preferred_element_type=jnp.float32)

    @pl.when(k == GK - 1)
    def _():
      r1 = _rsqrt0(od_i[...])
      r2 = _rsqrt0(idc_i[...])
      r3 = _rsqrt0(rso_i[...])
      r4 = _rsqrt0(lax.dot_general(csi_i[...], ones_g, dn_t,
                                   preferred_element_type=jnp.float32))
      us = C_F_SRC * (r1 * acc1[...]) + C_S_SRC * (r3 * acc3[...])
      ud = C_F_DST * (r2 * acc2[...]) + C_S_DST * (r4 * acc4[...])
      dn_wt = (((1,), (1,)), ((), ()))
      o = lax.dot_general(us, ws[...], dn_wt,
                          preferred_element_type=jnp.float32)
      o += lax.dot_general(ud, wd[...], dn_wt,
                           preferred_element_type=jnp.float32)
      out_ref[...] = o + C_BIAS * (bs[...] + bd[...])

  return pl.pallas_call(
      body,
      grid=(GI, GK),
      in_specs=[
          pl.BlockSpec((BI, KB), lambda i, k: (i, k)),
          pl.BlockSpec((KB, BI), lambda i, k: (k, i)),
          pl.BlockSpec((BI, KB), lambda i, k: (i, k)),
          pl.BlockSpec((KB, BI), lambda i, k: (k, i)),
          pl.BlockSpec((KB, D), lambda i, k: (k, 0)),
          pl.BlockSpec((KB, 1), lambda i, k: (k, 0)),
          pl.BlockSpec((KB, 1), lambda i, k: (k, 0)),
          pl.BlockSpec((GI, KB), lambda i, k: (0, k)),
          pl.BlockSpec((GI, BI), lambda i, k: (0, i)),
          pl.BlockSpec((BI, 1), lambda i, k: (i, 0)),
          pl.BlockSpec((BI, 1), lambda i, k: (i, 0)),
          pl.BlockSpec((BI, 1), lambda i, k: (i, 0)),
          pl.BlockSpec((KB, 1), lambda i, k: (k, 0)),
          pl.BlockSpec((D, D), lambda i, k: (0, 0)),
          pl.BlockSpec((D, D), lambda i, k: (0, 0)),
          pl.BlockSpec((1, D), lambda i, k: (0, 0)),
          pl.BlockSpec((1, D), lambda i, k: (0, 0)),
      ],
      out_specs=pl.BlockSpec((BI, D), lambda i, k: (i, 0)),
      out_shape=jax.ShapeDtypeStruct((NP, D), jnp.float32),
      scratch_shapes=[pltpu.VMEM((BI, D), jnp.float32) for _ in range(4)],
      compiler_params=pltpu.CompilerParams(
          dimension_semantics=("arbitrary", "arbitrary")),
  )(abf, abf, qo, qi, xp, idegc, odeg, csop, csip,
    odeg, idegc, rso, rsi,
    w_src, w_dst, b_src, b_dst)


def kernel(x, edge_index, W_src, b_src, W_dst, b_dst):
  rows = edge_index[0].astype(jnp.int32)
  cols = edge_index[1].astype(jnp.int32)
  pad = jnp.full((EP - E,), NP - 1, jnp.int32)
  rows_p = jnp.concatenate([rows, pad])
  cols_p = jnp.concatenate([cols, pad])

  a_flat = _sc_build_adjacency(rows_p, cols_p)
  a = a_flat.reshape(NP, NP)

  abf, odeg, ideg_row = _k2_cast_degrees(a)
  ideg_col = ideg_row.reshape(NP, 1)

  qi, rsi, csip = _k3_product(abf, transposed=True)
  qo, rso, csop = _k3_product(abf, transposed=False)
  csip = csip.reshape(GI, NP)
  csop = csop.reshape(GI, NP)

  xp = jnp.pad(x.astype(jnp.float32), ((0, NP - N), (0, 0)))
  out = _k4_aggregate(abf, qo, qi, xp, odeg, ideg_col, rso, csop, rsi,
                      csip, W_src, W_dst,
                      b_src.reshape(1, D), b_dst.reshape(1, D))
  return out[:N]


# speedup vs baseline: 1.8674x; 1.0198x over previous
import functools

import jax
import jax.numpy as jnp
from jax import lax
from jax.experimental import pallas as pl
from jax.experimental.pallas import tpu as pltpu
from jax.experimental.pallas import tpu_sc as plsc

N = 10000
NP = 10240
D = 128
E = 160000
EP = 163840
ALPHA1 = 0.5
ALPHA2 = 0.5
BETA1 = 0.7

C_F_SRC = BETA1 * ALPHA1
C_F_DST = BETA1 * (1.0 - ALPHA1)
C_S_SRC = (1.0 - BETA1) * ALPHA2
C_S_DST = (1.0 - BETA1) * (1.0 - ALPHA2)
C_BIAS = BETA1 * ALPHA1 + (1.0 - BETA1) * ALPHA2

NSUB = 16
STRIPE = NP * NP // NSUB
ZCH = 32768
NZ = STRIPE // ZCH
EPW = EP // NSUB
NB = EPW // 128

BP = 512
GP = NP // BP
BI = 1280
KB = 1280
GI = NP // BI
GK = NP // KB
R2 = 320


def _sc_build_adjacency(rows, cols):
  mesh = plsc.VectorSubcoreMesh(
      core_axis_name="c", subcore_axis_name="s", num_cores=1)

  @functools.partial(
      pl.kernel,
      out_type=jax.ShapeDtypeStruct((NP * NP,), jnp.float32),
      mesh=mesh,
      scratch_types=[
          pltpu.VMEM((ZCH,), jnp.float32),
          pltpu.VMEM((128,), jnp.int32),
          pltpu.VMEM((128,), jnp.int32),
          pltpu.VMEM((128,), jnp.int32),
          pltpu.VMEM((128,), jnp.float32),
          pltpu.SemaphoreType.DMA,
      ],
  )
  def build(rows_hbm, cols_hbm, a_hbm, zbuf, rbuf, cbuf, ibuf, ones, sem):
    wid = lax.axis_index("s")

    def zfill(i, c):
      zbuf[pl.ds(i * 16, 16)] = jnp.zeros((16,), jnp.float32)
      return c
    lax.fori_loop(0, ZCH // 16, zfill, 0)

    def ofill(i, c):
      ones[pl.ds(i * 16, 16)] = jnp.ones((16,), jnp.float32)
      return c
    lax.fori_loop(0, 8, ofill, 0)

    base0 = wid * STRIPE

    def memset(j, c):
      pltpu.sync_copy(zbuf, a_hbm.at[pl.ds(base0 + j * ZCH, ZCH)])
      return c
    lax.fori_loop(0, NZ, memset, 0)

    plsc.subcore_barrier()

    ebase = wid * EPW

    def scat(b, c):
      s = ebase + b * 128
      pltpu.sync_copy(rows_hbm.at[pl.ds(s, 128)], rbuf)
      pltpu.sync_copy(cols_hbm.at[pl.ds(s, 128)], cbuf)
      for i in range(8):
        rv = rbuf[pl.ds(i * 16, 16)]
        cv = cbuf[pl.ds(i * 16, 16)]
        ibuf[pl.ds(i * 16, 16)] = rv * NP + cv
      pltpu.async_copy(ones, a_hbm.at[ibuf], sem).wait()
      return c
    lax.fori_loop(0, NB, scat, 0)

  return build(rows, cols)


def _k2_cast_degrees(a):
  def body(a_ref, abf_ref, od_ref, idr_ref):
    blk = a_ref[...]
    abf_ref[...] = blk.astype(jnp.bfloat16)
    od_ref[...] = jnp.sum(blk, axis=1, keepdims=True)
    part = jnp.sum(blk, axis=0, keepdims=True)

    @pl.when(pl.program_id(0) == 0)
    def _():
      idr_ref[...] = part

    @pl.when(pl.program_id(0) != 0)
    def _():
      idr_ref[...] += part

  return pl.pallas_call(
      body,
      grid=(NP // R2,),
      in_specs=[pl.BlockSpec((R2, NP), lambda i: (i, 0))],
      out_specs=[
          pl.BlockSpec((R2, NP), lambda i: (i, 0)),
          pl.BlockSpec((R2, 1), lambda i: (i, 0)),
          pl.BlockSpec((1, NP), lambda i: (0, 0)),
      ],
      out_shape=[
          jax.ShapeDtypeStruct((NP, NP), jnp.bfloat16),
          jax.ShapeDtypeStruct((NP, 1), jnp.float32),
          jax.ShapeDtypeStruct((1, NP), jnp.float32),
      ],
      compiler_params=pltpu.CompilerParams(
          dimension_semantics=("arbitrary",)),
  )(a)


def _k3_product(abf, transposed):
  def body(l_ref, r_ref, m_ref, q_ref, rs_ref, cs_ref):
    i = pl.program_id(0)
    j = pl.program_id(1)

    if transposed:
      dn = (((0,), (0,)), ((), ()))
    else:
      dn = (((1,), (1,)), ((), ()))
    p = lax.dot_general(l_ref[...], r_ref[...], dn,
                        preferred_element_type=jnp.float32)
    amask = m_ref[...] > 0
    ii = lax.broadcasted_iota(jnp.int32, (BP, BP), 0)
    jj = lax.broadcasted_iota(jnp.int32, (BP, BP), 1)
    diag = (ii == jj) & (i == j)
    q = jnp.where(amask | diag, 0.0, p)
    q_ref[...] = q.astype(jnp.bfloat16)
    rpart = jnp.sum(q, axis=1, keepdims=True)

    @pl.when(j == 0)
    def _():
      rs_ref[...] = rpart

    @pl.when(j != 0)
    def _():
      rs_ref[...] += rpart

    cs_ref[...] = jnp.sum(q, axis=0, keepdims=True)[None]

  if transposed:
    lhs_spec = pl.BlockSpec((NP, BP), lambda i, j: (0, i))
    rhs_spec = pl.BlockSpec((NP, BP), lambda i, j: (0, j))
  else:
    lhs_spec = pl.BlockSpec((BP, NP), lambda i, j: (i, 0))
    rhs_spec = pl.BlockSpec((BP, NP), lambda i, j: (j, 0))

  return pl.pallas_call(
      body,
      grid=(GP, GP),
      in_specs=[
          lhs_spec,
          rhs_spec,
          pl.BlockSpec((BP, BP), lambda i, j: (i, j)),
      ],
      out_specs=[
          pl.BlockSpec((BP, BP), lambda i, j: (i, j)),
          pl.BlockSpec((BP, 1), lambda i, j: (i, 0)),
          pl.BlockSpec((1, 1, BP), lambda i, j: (i, 0, j)),
      ],
      out_shape=[
          jax.ShapeDtypeStruct((NP, NP), jnp.bfloat16),
          jax.ShapeDtypeStruct((NP, 1), jnp.float32),
          jax.ShapeDtypeStruct((GP, 1, NP), jnp.float32),
      ],
      compiler_params=pltpu.CompilerParams(
          dimension_semantics=("arbitrary", "arbitrary")),
  )(abf, abf, abf)


def _rsqrt0(v):
  return jnp.where(v > 0, lax.rsqrt(v), 0.0)


def _k4_aggregate(abf, qo, qi, xp, odeg, idegc, rso, csop, rsi, csip,
                  w_src, w_dst, b_src, b_dst):
  def body(a_ik, a_ki, qo_ik, qi_ki, xk, idc_k, od_k, cso_k, csi_i,
           od_i, idc_i, rso_i, rsi_k, ws, wd, bs, bd,
           out_ref, acc1, acc2, acc3, acc4):
    k = pl.program_id(1)

    @pl.when(k == 0)
    def _():
      acc1[...] = jnp.zeros_like(acc1)
      acc2[...] = jnp.zeros_like(acc2)
      acc3[...] = jnp.zeros_like(acc3)
      acc4[...] = jnp.zeros_like(acc4)

    ones_g = jnp.ones((GP, 1), jnp.float32)
    dn_t = (((0,), (0,)), ((), ()))
    dn_s = (((1,), (0,)), ((), ()))

    xb = xk[...]
    xs1 = (_rsqrt0(idc_k[...]) * xb).astype(jnp.bfloat16)
    xs2 = (_rsqrt0(od_k[...]) * xb).astype(jnp.bfloat16)
    c3 = _rsqrt0(lax.dot_general(cso_k[...], ones_g, dn_t,
                                 preferred_element_type=jnp.float32))
    xs3 = (c3 * xb).astype(jnp.bfloat16)
    c4 = _rsqrt0(rsi_k[...])
    xs4 = (c4 * xb).astype(jnp.bfloat16)

    acc1[...] += lax.dot_general(a_ik[...], xs1, dn_s,
                                 preferred_element_type=jnp.float32)
    acc2[...] += lax.dot_general(a_ki[...], xs2, dn_t,
                                 preferred_element_type=jnp.float32)
    acc3[...] += lax.dot_general(qo_ik[...], xs3, dn_s,
                                 preferred_element_type=jnp.float32)
    acc4[...] += lax.dot_general(qi_ki[...], xs4, dn_t,
                                 preferred_element_type=jnp.float32)

    @pl.when(k == GK - 1)
    def _():
      r1 = _rsqrt0(od_i[...])
      r2 = _rsqrt0(idc_i[...])
      r3 = _rsqrt0(rso_i[...])
      r4 = _rsqrt0(lax.dot_general(csi_i[...], ones_g, dn_t,
                                   preferred_element_type=jnp.float32))
      us = C_F_SRC * (r1 * acc1[...]) + C_S_SRC * (r3 * acc3[...])
      ud = C_F_DST * (r2 * acc2[...]) + C_S_DST * (r4 * acc4[...])
      dn_wt = (((1,), (1,)), ((), ()))
      o = lax.dot_general(us, ws[...], dn_wt,
                          preferred_element_type=jnp.float32)
      o += lax.dot_general(ud, wd[...], dn_wt,
                           preferred_element_type=jnp.float32)
      out_ref[...] = o + C_BIAS * (bs[...] + bd[...])

  return pl.pallas_call(
      body,
      grid=(GI, GK),
      in_specs=[
          pl.BlockSpec((BI, KB), lambda i, k: (i, k)),
          pl.BlockSpec((KB, BI), lambda i, k: (k, i)),
          pl.BlockSpec((BI, KB), lambda i, k: (i, k)),
          pl.BlockSpec((KB, BI), lambda i, k: (k, i)),
          pl.BlockSpec((KB, D), lambda i, k: (k, 0)),
          pl.BlockSpec((KB, 1), lambda i, k: (k, 0)),
          pl.BlockSpec((KB, 1), lambda i, k: (k, 0)),
          pl.BlockSpec((GP, KB), lambda i, k: (0, k)),
          pl.BlockSpec((GP, BI), lambda i, k: (0, i)),
          pl.BlockSpec((BI, 1), lambda i, k: (i, 0)),
          pl.BlockSpec((BI, 1), lambda i, k: (i, 0)),
          pl.BlockSpec((BI, 1), lambda i, k: (i, 0)),
          pl.BlockSpec((KB, 1), lambda i, k: (k, 0)),
          pl.BlockSpec((D, D), lambda i, k: (0, 0)),
          pl.BlockSpec((D, D), lambda i, k: (0, 0)),
          pl.BlockSpec((1, D), lambda i, k: (0, 0)),
          pl.BlockSpec((1, D), lambda i, k: (0, 0)),
      ],
      out_specs=pl.BlockSpec((BI, D), lambda i, k: (i, 0)),
      out_shape=jax.ShapeDtypeStruct((NP, D), jnp.float32),
      scratch_shapes=[pltpu.VMEM((BI, D), jnp.float32) for _ in range(4)],
      compiler_params=pltpu.CompilerParams(
          dimension_semantics=("arbitrary", "arbitrary")),
  )(abf, abf, qo, qi, xp, idegc, odeg, csop, csip,
    odeg, idegc, rso, rsi,
    w_src, w_dst, b_src, b_dst)


def kernel(x, edge_index, W_src, b_src, W_dst, b_dst):
  rows = edge_index[0].astype(jnp.int32)
  cols = edge_index[1].astype(jnp.int32)
  pad = jnp.full((EP - E,), NP - 1, jnp.int32)
  rows_p = jnp.concatenate([rows, pad])
  cols_p = jnp.concatenate([cols, pad])

  a_flat = _sc_build_adjacency(rows_p, cols_p)
  a = a_flat.reshape(NP, NP)

  abf, odeg, ideg_row = _k2_cast_degrees(a)
  ideg_col = ideg_row.reshape(NP, 1)

  qi, rsi, csip = _k3_product(abf, transposed=True)
  qo, rso, csop = _k3_product(abf, transposed=False)
  csip = csip.reshape(GP, NP)
  csop = csop.reshape(GP, NP)

  xp = jnp.pad(x.astype(jnp.float32), ((0, NP - N), (0, 0)))
  out = _k4_aggregate(abf, qo, qi, xp, odeg, ideg_col, rso, csop, rsi,
                      csip, W_src, W_dst,
                      b_src.reshape(1, D), b_dst.reshape(1, D))
  return out[:N]


# speedup vs baseline: 2.6013x; 1.3930x over previous
import functools

import jax
import jax.numpy as jnp
from jax import lax
from jax.experimental import pallas as pl
from jax.experimental.pallas import tpu as pltpu
from jax.experimental.pallas import tpu_sc as plsc

N = 10000
NP = 10240
D = 128
E = 160000
EP = 163840
ALPHA1 = 0.5
ALPHA2 = 0.5
BETA1 = 0.7

C_F_SRC = BETA1 * ALPHA1
C_F_DST = BETA1 * (1.0 - ALPHA1)
C_S_SRC = (1.0 - BETA1) * ALPHA2
C_S_DST = (1.0 - BETA1) * (1.0 - ALPHA2)
C_BIAS = BETA1 * ALPHA1 + (1.0 - BETA1) * ALPHA2

NSUB = 16
STRIPE = NP * NP // NSUB
ZCH = 32768
NZ = STRIPE // ZCH
EPW = EP // NSUB
NB = EPW // 128

BP = 512
GP = NP // BP
BI = 1280
KB = 1280
GI = NP // BI
GK = NP // KB
R2 = 320


def _sc_build_adjacency(rows, cols):
  mesh = plsc.VectorSubcoreMesh(
      core_axis_name="c", subcore_axis_name="s", num_cores=1)

  @functools.partial(
      pl.kernel,
      out_type=jax.ShapeDtypeStruct((NP * NP,), jnp.float32),
      mesh=mesh,
      scratch_types=[
          pltpu.VMEM((ZCH,), jnp.float32),
          pltpu.VMEM((128,), jnp.int32),
          pltpu.VMEM((128,), jnp.int32),
          pltpu.VMEM((128,), jnp.int32),
          pltpu.VMEM((128,), jnp.float32),
          pltpu.SemaphoreType.DMA,
      ],
  )
  def build(rows_hbm, cols_hbm, a_hbm, zbuf, rbuf, cbuf, ibuf, ones, sem):
    wid = lax.axis_index("s")

    def zfill(i, c):
      zbuf[pl.ds(i * 16, 16)] = jnp.zeros((16,), jnp.float32)
      return c
    lax.fori_loop(0, ZCH // 16, zfill, 0)

    def ofill(i, c):
      ones[pl.ds(i * 16, 16)] = jnp.ones((16,), jnp.float32)
      return c
    lax.fori_loop(0, 8, ofill, 0)

    base0 = wid * STRIPE

    def memset(j, c):
      pltpu.sync_copy(zbuf, a_hbm.at[pl.ds(base0 + j * ZCH, ZCH)])
      return c
    lax.fori_loop(0, NZ, memset, 0)

    plsc.subcore_barrier()

    ebase = wid * EPW

    def scat(b, c):
      s = ebase + b * 128
      pltpu.sync_copy(rows_hbm.at[pl.ds(s, 128)], rbuf)
      pltpu.sync_copy(cols_hbm.at[pl.ds(s, 128)], cbuf)
      for i in range(8):
        rv = rbuf[pl.ds(i * 16, 16)]
        cv = cbuf[pl.ds(i * 16, 16)]
        ibuf[pl.ds(i * 16, 16)] = rv * NP + cv
      pltpu.async_copy(ones, a_hbm.at[ibuf], sem).wait()
      return c
    lax.fori_loop(0, NB, scat, 0)

  return build(rows, cols)


def _k2_cast_degrees(a):
  def body(a_ref, abf_ref, od_ref, idr_ref):
    blk = a_ref[...]
    abf_ref[...] = blk.astype(jnp.bfloat16)
    od_ref[...] = jnp.sum(blk, axis=1, keepdims=True)
    part = jnp.sum(blk, axis=0, keepdims=True)

    @pl.when(pl.program_id(0) == 0)
    def _():
      idr_ref[...] = part

    @pl.when(pl.program_id(0) != 0)
    def _():
      idr_ref[...] += part

  return pl.pallas_call(
      body,
      grid=(NP // R2,),
      in_specs=[pl.BlockSpec((R2, NP), lambda i: (i, 0))],
      out_specs=[
          pl.BlockSpec((R2, NP), lambda i: (i, 0)),
          pl.BlockSpec((R2, 1), lambda i: (i, 0)),
          pl.BlockSpec((1, NP), lambda i: (0, 0)),
      ],
      out_shape=[
          jax.ShapeDtypeStruct((NP, NP), jnp.bfloat16),
          jax.ShapeDtypeStruct((NP, 1), jnp.float32),
          jax.ShapeDtypeStruct((1, NP), jnp.float32),
      ],
      compiler_params=pltpu.CompilerParams(
          dimension_semantics=("arbitrary",)),
  )(a)


def _tri_ij(t):
  c = 2 * GP + 1
  disc = (c * c - 8 * t).astype(jnp.float32)
  i = ((c - jnp.sqrt(disc)) * 0.5).astype(jnp.int32)
  base = i * GP - (i * (i - 1)) // 2
  j = t - base + i
  return i, j


def _k3_product(abf, transposed):
  TRI = GP * (GP + 1) // 2

  def body(l_ref, r_ref, m1_ref, m2_ref, _z1, _z2, _z3,
           qa_ref, qb_ref, rsa_ref, rsb_ref, csa_ref, csb_ref):
    t = pl.program_id(0)
    i, j = _tri_ij(t)

    if transposed:
      dn = (((0,), (0,)), ((), ()))
    else:
      dn = (((1,), (1,)), ((), ()))
    p = lax.dot_general(l_ref[...], r_ref[...], dn,
                        preferred_element_type=jnp.float32)

    ii = lax.broadcasted_iota(jnp.int32, (BP, BP), 0)
    jj = lax.broadcasted_iota(jnp.int32, (BP, BP), 1)
    diag = (ii == jj) & (i == j)
    q1 = jnp.where((m1_ref[...] > 0) | diag, 0.0, p)
    qa_ref[...] = q1.astype(jnp.bfloat16)
    rpart = jnp.sum(q1, axis=1, keepdims=True)

    @pl.when(j == i)
    def _():
      rsa_ref[...] = rpart

    @pl.when(j != i)
    def _():
      rsa_ref[...] += rpart

    csa_ref[...] = jnp.sum(q1, axis=0, keepdims=True)[None]

    pt = jnp.transpose(p)
    q2 = jnp.where((m2_ref[...] > 0) | (i == j), 0.0, pt)
    qb_ref[...] = q2.astype(jnp.bfloat16)
    rsb_ref[...] = jnp.transpose(jnp.sum(q2, axis=1, keepdims=True))[None]
    csb_ref[...] = jnp.sum(q2, axis=0, keepdims=True)[None]

  if transposed:
    lhs_spec = pl.BlockSpec((NP, BP), lambda t: (0, _tri_ij(t)[0]))
    rhs_spec = pl.BlockSpec((NP, BP), lambda t: (0, _tri_ij(t)[1]))
  else:
    lhs_spec = pl.BlockSpec((BP, NP), lambda t: (_tri_ij(t)[0], 0))
    rhs_spec = pl.BlockSpec((BP, NP), lambda t: (_tri_ij(t)[1], 0))

  zpart = jnp.zeros((GP, 1, NP), jnp.float32)
  return pl.pallas_call(
      body,
      grid=(TRI,),
      in_specs=[
          lhs_spec,
          rhs_spec,
          pl.BlockSpec((BP, BP), lambda t: _tri_ij(t)),
          pl.BlockSpec((BP, BP), lambda t: _tri_ij(t)[::-1]),
          pl.BlockSpec((1, 1, BP),
                       lambda t: (_tri_ij(t)[0], 0, _tri_ij(t)[1])),
          pl.BlockSpec((1, 1, BP),
                       lambda t: (_tri_ij(t)[0], 0, _tri_ij(t)[1])),
          pl.BlockSpec((1, 1, BP),
                       lambda t: (_tri_ij(t)[1], 0, _tri_ij(t)[0])),
      ],
      out_specs=[
          pl.BlockSpec((BP, BP), lambda t: _tri_ij(t)),
          pl.BlockSpec((BP, BP), lambda t: _tri_ij(t)[::-1]),
          pl.BlockSpec((BP, 1), lambda t: (_tri_ij(t)[0], 0)),
          pl.BlockSpec((1, 1, BP),
                       lambda t: (_tri_ij(t)[0], 0, _tri_ij(t)[1])),
          pl.BlockSpec((1, 1, BP),
                       lambda t: (_tri_ij(t)[0], 0, _tri_ij(t)[1])),
          pl.BlockSpec((1, 1, BP),
                       lambda t: (_tri_ij(t)[1], 0, _tri_ij(t)[0])),
      ],
      out_shape=[
          jax.ShapeDtypeStruct((NP, NP), jnp.bfloat16),
          jax.ShapeDtypeStruct((NP, NP), jnp.bfloat16),
          jax.ShapeDtypeStruct((NP, 1), jnp.float32),
          jax.ShapeDtypeStruct((GP, 1, NP), jnp.float32),
          jax.ShapeDtypeStruct((GP, 1, NP), jnp.float32),
          jax.ShapeDtypeStruct((GP, 1, NP), jnp.float32),
      ],
      input_output_aliases={4: 3, 5: 4, 6: 5},
      compiler_params=pltpu.CompilerParams(
          dimension_semantics=("arbitrary",)),
  )(abf, abf, abf, abf, zpart, zpart, zpart)


def _rsqrt0(v):
  return jnp.where(v > 0, lax.rsqrt(v), 0.0)


def _k4_aggregate(abf, qoA, qoB, qiA, qiB, xp, odeg, idegc,
                  rsoA, rsoBp, csoAp, csoBp, rsiA, rsiBp, csiAp, csiBp,
                  w_src, w_dst, b_src, b_dst):
  def body(a_ik, a_ki, qoa, qob, qia, qib, xk, idc_k, od_k,
           csoA_k, csoB_k, rsiA_k, rsiB_k,
           od_i, idc_i, rsoA_i, rsoB_i, csiA_i, csiB_i,
           ws, wd, bs, bd,
           out_ref, acc1, acc2, acc3, acc4):
    i = pl.program_id(0)
    k = pl.program_id(1)

    @pl.when(k == 0)
    def _():
      acc1[...] = jnp.zeros_like(acc1)
      acc2[...] = jnp.zeros_like(acc2)
      acc3[...] = jnp.zeros_like(acc3)
      acc4[...] = jnp.zeros_like(acc4)

    ones_g = jnp.ones((GP, 1), jnp.float32)
    dn_t = (((0,), (0,)), ((), ()))
    dn_s = (((1,), (0,)), ((), ()))

    ir = lax.broadcasted_iota(jnp.int32, (BI, KB), 0)
    ic = lax.broadcasted_iota(jnp.int32, (BI, KB), 1)
    sel_ik = ((i * BI + ir) // BP) <= ((k * KB + ic) // BP)
    qo_t = jnp.where(sel_ik, qoa[...], qob[...])
    sel_ki = ((k * KB + ir) // BP) <= ((i * BI + ic) // BP)
    qi_t = jnp.where(sel_ki, qia[...], qib[...])

    xb = xk[...]
    xs1 = (_rsqrt0(idc_k[...]) * xb).astype(jnp.bfloat16)
    xs2 = (_rsqrt0(od_k[...]) * xb).astype(jnp.bfloat16)
    c3 = _rsqrt0(
        lax.dot_general(csoA_k[...], ones_g, dn_t,
                        preferred_element_type=jnp.float32)
        + lax.dot_general(csoB_k[...], ones_g, dn_t,
                          preferred_element_type=jnp.float32))
    xs3 = (c3 * xb).astype(jnp.bfloat16)
    c4 = _rsqrt0(rsiA_k[...]
                 + lax.dot_general(rsiB_k[...], ones_g, dn_t,
                                   preferred_element_type=jnp.float32))
    xs4 = (c4 * xb).astype(jnp.bfloat16)

    acc1[...] += lax.dot_general(a_ik[...], xs1, dn_s,
                                 preferred_element_type=jnp.float32)
    acc2[...] += lax.dot_general(a_ki[...], xs2, dn_t,
                                 preferred_element_type=jnp.float32)
    acc3[...] += lax.dot_general(qo_t, xs3, dn_s,
                                 preferred_element_type=jnp.float32)
    acc4[...] += lax.dot_general(qi_t, xs4, dn_t,
                                 preferred_element_type=jnp.float32)

    @pl.when(k == GK - 1)
    def _():
      r1 = _rsqrt0(od_i[...])
      r2 = _rsqrt0(idc_i[...])
      r3 = _rsqrt0(rsoA_i[...]
                   + lax.dot_general(rsoB_i[...], ones_g, dn_t,
                                     preferred_element_type=jnp.float32))
      r4 = _rsqrt0(
          lax.dot_general(csiA_i[...], ones_g, dn_t,
                          preferred_element_type=jnp.float32)
          + lax.dot_general(csiB_i[...], ones_g, dn_t,
                            preferred_element_type=jnp.float32))
      us = C_F_SRC * (r1 * acc1[...]) + C_S_SRC * (r3 * acc3[...])
      ud = C_F_DST * (r2 * acc2[...]) + C_S_DST * (r4 * acc4[...])
      dn_wt = (((1,), (1,)), ((), ()))
      o = lax.dot_general(us, ws[...], dn_wt,
                          preferred_element_type=jnp.float32)
      o += lax.dot_general(ud, wd[...], dn_wt,
                           preferred_element_type=jnp.float32)
      out_ref[...] = o + C_BIAS * (bs[...] + bd[...])

  return pl.pallas_call(
      body,
      grid=(GI, GK),
      in_specs=[
          pl.BlockSpec((BI, KB), lambda i, k: (i, k)),
          pl.BlockSpec((KB, BI), lambda i, k: (k, i)),
          pl.BlockSpec((BI, KB), lambda i, k: (i, k)),
          pl.BlockSpec((BI, KB), lambda i, k: (i, k)),
          pl.BlockSpec((KB, BI), lambda i, k: (k, i)),
          pl.BlockSpec((KB, BI), lambda i, k: (k, i)),
          pl.BlockSpec((KB, D), lambda i, k: (k, 0)),
          pl.BlockSpec((KB, 1), lambda i, k: (k, 0)),
          pl.BlockSpec((KB, 1), lambda i, k: (k, 0)),
          pl.BlockSpec((GP, KB), lambda i, k: (0, k)),
          pl.BlockSpec((GP, KB), lambda i, k: (0, k)),
          pl.BlockSpec((KB, 1), lambda i, k: (k, 0)),
          pl.BlockSpec((GP, KB), lambda i, k: (0, k)),
          pl.BlockSpec((BI, 1), lambda i, k: (i, 0)),
          pl.BlockSpec((BI, 1), lambda i, k: (i, 0)),
          pl.BlockSpec((BI, 1), lambda i, k: (i, 0)),
          pl.BlockSpec((GP, BI), lambda i, k: (0, i)),
          pl.BlockSpec((GP, BI), lambda i, k: (0, i)),
          pl.BlockSpec((GP, BI), lambda i, k: (0, i)),
          pl.BlockSpec((D, D), lambda i, k: (0, 0)),
          pl.BlockSpec((D, D), lambda i, k: (0, 0)),
          pl.BlockSpec((1, D), lambda i, k: (0, 0)),
          pl.BlockSpec((1, D), lambda i, k: (0, 0)),
      ],
      out_specs=pl.BlockSpec((BI, D), lambda i, k: (i, 0)),
      out_shape=jax.ShapeDtypeStruct((NP, D), jnp.float32),
      scratch_shapes=[pltpu.VMEM((BI, D), jnp.float32) for _ in range(4)],
      compiler_params=pltpu.CompilerParams(
          dimension_semantics=("arbitrary", "arbitrary")),
  )(abf, abf, qoA, qoB, qiA, qiB, xp, idegc, odeg,
    csoAp, csoBp, rsiA, rsiBp,
    odeg, idegc, rsoA, rsoBp, csiAp, csiBp,
    w_src, w_dst, b_src, b_dst)


def kernel(x, edge_index, W_src, b_src, W_dst, b_dst):
  rows = edge_index[0].astype(jnp.int32)
  cols = edge_index[1].astype(jnp.int32)
  pad = jnp.full((EP - E,), NP - 1, jnp.int32)
  rows_p = jnp.concatenate([rows, pad])
  cols_p = jnp.concatenate([cols, pad])

  a_flat = _sc_build_adjacency(rows_p, cols_p)
  a = a_flat.reshape(NP, NP)

  abf, odeg, ideg_row = _k2_cast_degrees(a)
  ideg_col = ideg_row.reshape(NP, 1)

  qiA, qiB, rsiA, rsiBp, csiAp, csiBp = _k3_product(abf, transposed=True)
  qoA, qoB, rsoA, rsoBp, csoAp, csoBp = _k3_product(abf, transposed=False)
  rsiBp = rsiBp.reshape(GP, NP)
  csiAp = csiAp.reshape(GP, NP)
  csiBp = csiBp.reshape(GP, NP)
  rsoBp = rsoBp.reshape(GP, NP)
  csoAp = csoAp.reshape(GP, NP)
  csoBp = csoBp.reshape(GP, NP)

  xp = jnp.pad(x.astype(jnp.float32), ((0, NP - N), (0, 0)))
  out = _k4_aggregate(abf, qoA, qoB, qiA, qiB, xp, odeg, ideg_col,
                      rsoA, rsoBp, csoAp, csoBp, rsiA, rsiBp, csiAp, csiBp,
                      W_src, W_dst,
                      b_src.reshape(1, D), b_dst.reshape(1, D))
  return out[:N]


# speedup vs baseline: 2.6064x; 1.0020x over previous
import functools

import jax
import jax.numpy as jnp
from jax import lax
from jax.experimental import pallas as pl
from jax.experimental.pallas import tpu as pltpu
from jax.experimental.pallas import tpu_sc as plsc

N = 10000
NP = 10240
D = 128
E = 160000
EP = 163840
ALPHA1 = 0.5
ALPHA2 = 0.5
BETA1 = 0.7

C_F_SRC = BETA1 * ALPHA1
C_F_DST = BETA1 * (1.0 - ALPHA1)
C_S_SRC = (1.0 - BETA1) * ALPHA2
C_S_DST = (1.0 - BETA1) * (1.0 - ALPHA2)
C_BIAS = BETA1 * ALPHA1 + (1.0 - BETA1) * ALPHA2

NSUB = 16
STRIPE = NP * NP // NSUB
ZCH = 32768
NZ = STRIPE // ZCH
EPW = EP // NSUB
NB = EPW // 128

BP = 512
GP = NP // BP
BI = 1280
KB = 1280
GI = NP // BI
GK = NP // KB
R2 = 320


def _sc_build_adjacency(rows, cols):
  mesh = plsc.VectorSubcoreMesh(
      core_axis_name="c", subcore_axis_name="s", num_cores=1)

  @functools.partial(
      pl.kernel,
      out_type=jax.ShapeDtypeStruct((NP * NP,), jnp.float32),
      mesh=mesh,
      scratch_types=[
          pltpu.VMEM((ZCH,), jnp.float32),
          pltpu.VMEM((EPW,), jnp.int32),
          pltpu.VMEM((EPW,), jnp.int32),
          pltpu.VMEM((NB, 128), jnp.int32),
          pltpu.VMEM((128,), jnp.float32),
          pltpu.SemaphoreType.DMA,
      ],
  )
  def build(rows_hbm, cols_hbm, a_hbm, zbuf, rbuf, cbuf, ibuf, ones, sem):
    wid = lax.axis_index("s")

    def zfill(i, c):
      zbuf[pl.ds(i * 16, 16)] = jnp.zeros((16,), jnp.float32)
      return c
    lax.fori_loop(0, ZCH // 16, zfill, 0)

    def ofill(i, c):
      ones[pl.ds(i * 16, 16)] = jnp.ones((16,), jnp.float32)
      return c
    lax.fori_loop(0, 8, ofill, 0)

    base0 = wid * STRIPE
    ebase = wid * EPW
    pltpu.sync_copy(rows_hbm.at[pl.ds(ebase, EPW)], rbuf)
    pltpu.sync_copy(cols_hbm.at[pl.ds(ebase, EPW)], cbuf)

    def mset(j, c):
      pltpu.async_copy(zbuf, a_hbm.at[pl.ds(base0 + j * ZCH, ZCH)], sem)

      @pl.when(j >= 8)
      def _():
        pltpu.make_async_copy(
            zbuf, a_hbm.at[pl.ds(base0 + (j - 8) * ZCH, ZCH)], sem).wait()
      return c
    lax.fori_loop(0, NZ, mset, 0)

    def mdrain(j, c):
      pltpu.make_async_copy(
          zbuf, a_hbm.at[pl.ds(base0 + (NZ - 8 + j) * ZCH, ZCH)], sem).wait()
      return c
    lax.fori_loop(0, 8, mdrain, 0)

    plsc.subcore_barrier()

    def icomp(b, c):
      for l in range(8):
        rv = rbuf[pl.ds(b * 128 + l * 16, 16)]
        cv = cbuf[pl.ds(b * 128 + l * 16, 16)]
        ibuf[b, pl.ds(l * 16, 16)] = rv * NP + cv
      return c
    lax.fori_loop(0, NB, icomp, 0)

    def sfire(b, c):
      pltpu.async_copy(ones, a_hbm.at[ibuf.at[b]], sem)

      @pl.when(b >= 8)
      def _():
        pltpu.make_async_copy(ones, a_hbm.at[ibuf.at[b - 8]], sem).wait()
      return c
    lax.fori_loop(0, NB, sfire, 0)

    def sdrain(b, c):
      pltpu.make_async_copy(ones, a_hbm.at[ibuf.at[NB - 8 + b]], sem).wait()
      return c
    lax.fori_loop(0, 8, sdrain, 0)

  return build(rows, cols)


def _k2_cast_degrees(a):
  def body(a_ref, abf_ref, od_ref, idr_ref):
    blk = a_ref[...]
    abf_ref[...] = blk.astype(jnp.bfloat16)
    od_ref[...] = jnp.sum(blk, axis=1, keepdims=True)
    part = jnp.sum(blk, axis=0, keepdims=True)

    @pl.when(pl.program_id(0) == 0)
    def _():
      idr_ref[...] = part

    @pl.when(pl.program_id(0) != 0)
    def _():
      idr_ref[...] += part

  return pl.pallas_call(
      body,
      grid=(NP // R2,),
      in_specs=[pl.BlockSpec((R2, NP), lambda i: (i, 0))],
      out_specs=[
          pl.BlockSpec((R2, NP), lambda i: (i, 0)),
          pl.BlockSpec((R2, 1), lambda i: (i, 0)),
          pl.BlockSpec((1, NP), lambda i: (0, 0)),
      ],
      out_shape=[
          jax.ShapeDtypeStruct((NP, NP), jnp.bfloat16),
          jax.ShapeDtypeStruct((NP, 1), jnp.float32),
          jax.ShapeDtypeStruct((1, NP), jnp.float32),
      ],
      compiler_params=pltpu.CompilerParams(
          dimension_semantics=("arbitrary",)),
  )(a)


def _tri_ij(t):
  c = 2 * GP + 1
  disc = (c * c - 8 * t).astype(jnp.float32)
  i = ((c - jnp.sqrt(disc)) * 0.5).astype(jnp.int32)
  base = i * GP - (i * (i - 1)) // 2
  j = t - base + i
  return i, j


def _k3_product(abf, transposed):
  TRI = GP * (GP + 1) // 2

  def body(l_ref, r_ref, m1_ref, m2_ref, _z1, _z2, _z3,
           qa_ref, qb_ref, rsa_ref, rsb_ref, csa_ref, csb_ref):
    t = pl.program_id(0)
    i, j = _tri_ij(t)

    if transposed:
      dn = (((0,), (0,)), ((), ()))
    else:
      dn = (((1,), (1,)), ((), ()))
    p = lax.dot_general(l_ref[...], r_ref[...], dn,
                        preferred_element_type=jnp.float32)

    ii = lax.broadcasted_iota(jnp.int32, (BP, BP), 0)
    jj = lax.broadcasted_iota(jnp.int32, (BP, BP), 1)
    diag = (ii == jj) & (i == j)
    q1 = jnp.where((m1_ref[...] > 0) | diag, 0.0, p)
    qa_ref[...] = q1.astype(jnp.bfloat16)
    rpart = jnp.sum(q1, axis=1, keepdims=True)

    @pl.when(j == i)
    def _():
      rsa_ref[...] = rpart

    @pl.when(j != i)
    def _():
      rsa_ref[...] += rpart

    csa_ref[...] = jnp.sum(q1, axis=0, keepdims=True)[None]

    pt = jnp.transpose(p)
    q2 = jnp.where((m2_ref[...] > 0) | (i == j), 0.0, pt)
    qb_ref[...] = q2.astype(jnp.bfloat16)
    rsb_ref[...] = jnp.transpose(jnp.sum(q2, axis=1, keepdims=True))[None]
    csb_ref[...] = jnp.sum(q2, axis=0, keepdims=True)[None]

  if transposed:
    lhs_spec = pl.BlockSpec((NP, BP), lambda t: (0, _tri_ij(t)[0]))
    rhs_spec = pl.BlockSpec((NP, BP), lambda t: (0, _tri_ij(t)[1]))
  else:
    lhs_spec = pl.BlockSpec((BP, NP), lambda t: (_tri_ij(t)[0], 0))
    rhs_spec = pl.BlockSpec((BP, NP), lambda t: (_tri_ij(t)[1], 0))

  zpart = jnp.zeros((GP, 1, NP), jnp.float32)
  return pl.pallas_call(
      body,
      grid=(TRI,),
      in_specs=[
          lhs_spec,
          rhs_spec,
          pl.BlockSpec((BP, BP), lambda t: _tri_ij(t)),
          pl.BlockSpec((BP, BP), lambda t: _tri_ij(t)[::-1]),
          pl.BlockSpec((1, 1, BP),
                       lambda t: (_tri_ij(t)[0], 0, _tri_ij(t)[1])),
          pl.BlockSpec((1, 1, BP),
                       lambda t: (_tri_ij(t)[0], 0, _tri_ij(t)[1])),
          pl.BlockSpec((1, 1, BP),
                       lambda t: (_tri_ij(t)[1], 0, _tri_ij(t)[0])),
      ],
      out_specs=[
          pl.BlockSpec((BP, BP), lambda t: _tri_ij(t)),
          pl.BlockSpec((BP, BP), lambda t: _tri_ij(t)[::-1]),
          pl.BlockSpec((BP, 1), lambda t: (_tri_ij(t)[0], 0)),
          pl.BlockSpec((1, 1, BP),
                       lambda t: (_tri_ij(t)[0], 0, _tri_ij(t)[1])),
          pl.BlockSpec((1, 1, BP),
                       lambda t: (_tri_ij(t)[0], 0, _tri_ij(t)[1])),
          pl.BlockSpec((1, 1, BP),
                       lambda t: (_tri_ij(t)[1], 0, _tri_ij(t)[0])),
      ],
      out_shape=[
          jax.ShapeDtypeStruct((NP, NP), jnp.bfloat16),
          jax.ShapeDtypeStruct((NP, NP), jnp.bfloat16),
          jax.ShapeDtypeStruct((NP, 1), jnp.float32),
          jax.ShapeDtypeStruct((GP, 1, NP), jnp.float32),
          jax.ShapeDtypeStruct((GP, 1, NP), jnp.float32),
          jax.ShapeDtypeStruct((GP, 1, NP), jnp.float32),
      ],
      input_output_aliases={4: 3, 5: 4, 6: 5},
      compiler_params=pltpu.CompilerParams(
          dimension_semantics=("arbitrary",)),
  )(abf, abf, abf, abf, zpart, zpart, zpart)


def _rsqrt0(v):
  return jnp.where(v > 0, lax.rsqrt(v), 0.0)


def _k4_aggregate(abf, qoA, qoB, qiA, qiB, xp, odeg, idegc,
                  rsoA, rsoBp, csoAp, csoBp, rsiA, rsiBp, csiAp, csiBp,
                  w_src, w_dst, b_src, b_dst):
  def body(a_ik, a_ki, qoa, qob, qia, qib, xk, idc_k, od_k,
           csoA_k, csoB_k, rsiA_k, rsiB_k,
           od_i, idc_i, rsoA_i, rsoB_i, csiA_i, csiB_i,
           ws, wd, bs, bd,
           out_ref, acc1, acc2, acc3, acc4):
    i = pl.program_id(0)
    k = pl.program_id(1)

    @pl.when(k == 0)
    def _():
      acc1[...] = jnp.zeros_like(acc1)
      acc2[...] = jnp.zeros_like(acc2)
      acc3[...] = jnp.zeros_like(acc3)
      acc4[...] = jnp.zeros_like(acc4)

    ones_g = jnp.ones((GP, 1), jnp.float32)
    dn_t = (((0,), (0,)), ((), ()))
    dn_s = (((1,), (0,)), ((), ()))

    ir = lax.broadcasted_iota(jnp.int32, (BI, KB), 0)
    ic = lax.broadcasted_iota(jnp.int32, (BI, KB), 1)
    sel_ik = ((i * BI + ir) // BP) <= ((k * KB + ic) // BP)
    qo_t = jnp.where(sel_ik, qoa[...], qob[...])
    sel_ki = ((k * KB + ir) // BP) <= ((i * BI + ic) // BP)
    qi_t = jnp.where(sel_ki, qia[...], qib[...])

    xb = xk[...]
    xs1 = (_rsqrt0(idc_k[...]) * xb).astype(jnp.bfloat16)
    xs2 = (_rsqrt0(od_k[...]) * xb).astype(jnp.bfloat16)
    c3 = _rsqrt0(
        lax.dot_general(csoA_k[...], ones_g, dn_t,
                        preferred_element_type=jnp.float32)
        + lax.dot_general(csoB_k[...], ones_g, dn_t,
                          preferred_element_type=jnp.float32))
    xs3 = (c3 * xb).astype(jnp.bfloat16)
    c4 = _rsqrt0(rsiA_k[...]
                 + lax.dot_general(rsiB_k[...], ones_g, dn_t,
                                   preferred_element_type=jnp.float32))
    xs4 = (c4 * xb).astype(jnp.bfloat16)

    acc1[...] += lax.dot_general(a_ik[...], xs1, dn_s,
                                 preferred_element_type=jnp.float32)
    acc2[...] += lax.dot_general(a_ki[...], xs2, dn_t,
                                 preferred_element_type=jnp.float32)
    acc3[...] += lax.dot_general(qo_t, xs3, dn_s,
                                 preferred_element_type=jnp.float32)
    acc4[...] += lax.dot_general(qi_t, xs4, dn_t,
                                 preferred_element_type=jnp.float32)

    @pl.when(k == GK - 1)
    def _():
      r1 = _rsqrt0(od_i[...])
      r2 = _rsqrt0(idc_i[...])
      r3 = _rsqrt0(rsoA_i[...]
                   + lax.dot_general(rsoB_i[...], ones_g, dn_t,
                                     preferred_element_type=jnp.float32))
      r4 = _rsqrt0(
          lax.dot_general(csiA_i[...], ones_g, dn_t,
                          preferred_element_type=jnp.float32)
          + lax.dot_general(csiB_i[...], ones_g, dn_t,
                            preferred_element_type=jnp.float32))
      us = C_F_SRC * (r1 * acc1[...]) + C_S_SRC * (r3 * acc3[...])
      ud = C_F_DST * (r2 * acc2[...]) + C_S_DST * (r4 * acc4[...])
      dn_wt = (((1,), (1,)), ((), ()))
      o = lax.dot_general(us, ws[...], dn_wt,
                          preferred_element_type=jnp.float32)
      o += lax.dot_general(ud, wd[...], dn_wt,
                           preferred_element_type=jnp.float32)
      out_ref[...] = o + C_BIAS * (bs[...] + bd[...])

  return pl.pallas_call(
      body,
      grid=(GI, GK),
      in_specs=[
          pl.BlockSpec((BI, KB), lambda i, k: (i, k)),
          pl.BlockSpec((KB, BI), lambda i, k: (k, i)),
          pl.BlockSpec((BI, KB), lambda i, k: (i, k)),
          pl.BlockSpec((BI, KB), lambda i, k: (i, k)),
          pl.BlockSpec((KB, BI), lambda i, k: (k, i)),
          pl.BlockSpec((KB, BI), lambda i, k: (k, i)),
          pl.BlockSpec((KB, D), lambda i, k: (k, 0)),
          pl.BlockSpec((KB, 1), lambda i, k: (k, 0)),
          pl.BlockSpec((KB, 1), lambda i, k: (k, 0)),
          pl.BlockSpec((GP, KB), lambda i, k: (0, k)),
          pl.BlockSpec((GP, KB), lambda i, k: (0, k)),
          pl.BlockSpec((KB, 1), lambda i, k: (k, 0)),
          pl.BlockSpec((GP, KB), lambda i, k: (0, k)),
          pl.BlockSpec((BI, 1), lambda i, k: (i, 0)),
          pl.BlockSpec((BI, 1), lambda i, k: (i, 0)),
          pl.BlockSpec((BI, 1), lambda i, k: (i, 0)),
          pl.BlockSpec((GP, BI), lambda i, k: (0, i)),
          pl.BlockSpec((GP, BI), lambda i, k: (0, i)),
          pl.BlockSpec((GP, BI), lambda i, k: (0, i)),
          pl.BlockSpec((D, D), lambda i, k: (0, 0)),
          pl.BlockSpec((D, D), lambda i, k: (0, 0)),
          pl.BlockSpec((1, D), lambda i, k: (0, 0)),
          pl.BlockSpec((1, D), lambda i, k: (0, 0)),
      ],
      out_specs=pl.BlockSpec((BI, D), lambda i, k: (i, 0)),
      out_shape=jax.ShapeDtypeStruct((NP, D), jnp.float32),
      scratch_shapes=[pltpu.VMEM((BI, D), jnp.float32) for _ in range(4)],
      compiler_params=pltpu.CompilerParams(
          dimension_semantics=("arbitrary", "arbitrary")),
  )(abf, abf, qoA, qoB, qiA, qiB, xp, idegc, odeg,
    csoAp, csoBp, rsiA, rsiBp,
    odeg, idegc, rsoA, rsoBp, csiAp, csiBp,
    w_src, w_dst, b_src, b_dst)


def kernel(x, edge_index, W_src, b_src, W_dst, b_dst):
  rows = edge_index[0].astype(jnp.int32)
  cols = edge_index[1].astype(jnp.int32)
  pad = jnp.full((EP - E,), NP - 1, jnp.int32)
  rows_p = jnp.concatenate([rows, pad])
  cols_p = jnp.concatenate([cols, pad])

  a_flat = _sc_build_adjacency(rows_p, cols_p)
  a = a_flat.reshape(NP, NP)

  abf, odeg, ideg_row = _k2_cast_degrees(a)
  ideg_col = ideg_row.reshape(NP, 1)

  qiA, qiB, rsiA, rsiBp, csiAp, csiBp = _k3_product(abf, transposed=True)
  qoA, qoB, rsoA, rsoBp, csoAp, csoBp = _k3_product(abf, transposed=False)
  rsiBp = rsiBp.reshape(GP, NP)
  csiAp = csiAp.reshape(GP, NP)
  csiBp = csiBp.reshape(GP, NP)
  rsoBp = rsoBp.reshape(GP, NP)
  csoAp = csoAp.reshape(GP, NP)
  csoBp = csoBp.reshape(GP, NP)

  xp = jnp.pad(x.astype(jnp.float32), ((0, NP - N), (0, 0)))
  out = _k4_aggregate(abf, qoA, qoB, qiA, qiB, xp, odeg, ideg_col,
                      rsoA, rsoBp, csoAp, csoBp, rsiA, rsiBp, csiAp, csiBp,
                      W_src, W_dst,
                      b_src.reshape(1, D), b_dst.reshape(1, D))
  return out[:N]
